# baseline (speedup 1.0000x reference)
import functools

import jax
import jax.numpy as jnp
from jax import lax
from jax.experimental import pallas as pl
from jax.experimental.pallas import tpu as pltpu
from jax.experimental.pallas import tpu_sc as plsc

N = 10000
E = 160000
D_IN = 256
H = 128

NC = 2
NS = 16
NW = NC * NS
CH = 128
NCH = 40
EPT = CH * NCH
EPAD = NW * EPT
PCH = 128
NCH0 = 40
NCH1 = 40
NCHX = max(NCH0, NCH1, 1)
EPADP = NS * (NCH0 + NCH1) * PCH
NBUF = 2
SPAN = 632
ACCR = NS * SPAN
DSPAN = 640
DEGR = NS * DSPAN
TRASH = N

BLK = 1000
GRID = N // BLK

_sc_mesh = plsc.VectorSubcoreMesh(core_axis_name="c", subcore_axis_name="s")


@functools.partial(
    pl.kernel,
    out_type=jax.ShapeDtypeStruct((NC, DEGR), jnp.float32),
    mesh=_sc_mesh,
    scratch_types=[
        pltpu.VMEM((NCH, CH), jnp.int32),
        pltpu.VMEM((DSPAN,), jnp.float32),
        pltpu.VMEM((CH,), jnp.float32),
        pltpu.VMEM_SHARED((DEGR,), jnp.float32),
    ],
)
def _degree(dstp, out, dstv, buf, ones, deg):
    c = lax.axis_index("c")
    s = lax.axis_index("s")
    w = c * NS + s

    for i in range(DSPAN // 16):
        buf[pl.ds(i * 16, 16)] = jnp.zeros((16,), jnp.float32)
    for i in range(CH // 16):
        ones[pl.ds(i * 16, 16)] = jnp.ones((16,), jnp.float32)

    base = s * DSPAN
    pltpu.sync_copy(buf, deg.at[pl.ds(base, DSPAN)])
    plsc.subcore_barrier()

    pltpu.sync_copy(dstp.at[w], dstv)

    @pl.loop(0, NCH)
    def _(j):
        pltpu.sync_copy(ones, deg.at[dstv.at[j]], add=True)

    plsc.subcore_barrier()
    pltpu.sync_copy(deg.at[pl.ds(base, DSPAN)], buf)
    pltpu.sync_copy(buf, out.at[c, pl.ds(base, DSPAN)])


@functools.partial(
    pl.kernel,
    out_type=jax.ShapeDtypeStruct((NC, ACCR, H), jnp.float32),
    mesh=_sc_mesh,
    scratch_types=[
        pltpu.VMEM((NCHX, PCH), jnp.int32),
        pltpu.VMEM((NCHX, PCH), jnp.int32),
        pltpu.VMEM((PCH, H), jnp.float32),
        pltpu.VMEM((PCH, H), jnp.float32),
        pltpu.VMEM_SHARED((ACCR, H), jnp.float32),
        pltpu.SemaphoreType.DMA,
        pltpu.SemaphoreType.DMA,
    ],
)
def _propagate(hw, srcc0, dstc0, srcc1, dstc1, out,
               srcv, dstv, r0, r1, acc, gsem, ssem):
    rows = (r0, r1)
    c = lax.axis_index("c")
    s = lax.axis_index("s")

    with jax.named_scope("zero_acc"):
        @pl.loop(0, PCH)
        def _(r):
            for k in range(H // 16):
                rows[0][r, pl.ds(k * 16, 16)] = jnp.zeros((16,), jnp.float32)

        base = s * SPAN
        pieces = [(k * PCH, PCH) for k in range(SPAN // PCH)]
        if SPAN % PCH:
            pieces.append((SPAN - SPAN % PCH, SPAN % PCH))
        for off, ln in pieces:
            pltpu.async_copy(rows[0].at[pl.ds(0, ln)],
                             acc.at[pl.ds(base + off, ln)], ssem)
        for off, ln in pieces:
            pltpu.make_async_copy(rows[0].at[pl.ds(0, ln)],
                                  acc.at[pl.ds(base + off, ln)], ssem).wait()
        plsc.subcore_barrier()

    def run(srcc, dstc, cnt):
        pltpu.sync_copy(srcc.at[s], srcv.at[pl.ds(0, cnt)])
        pltpu.sync_copy(dstc.at[s], dstv.at[pl.ds(0, cnt)])

        def gstart(j, buf):
            pltpu.async_copy(hw.at[srcv.at[j]], buf, gsem)

        def gwait(j, buf):
            pltpu.make_async_copy(hw.at[srcv.at[j]], buf, gsem).wait()

        def sstart(j, buf):
            pltpu.async_copy(buf, acc.at[dstv.at[j]], ssem, add=True)

        def swait(j, buf):
            pltpu.make_async_copy(buf, acc.at[dstv.at[j]], ssem).wait()

        def step(j, swait_j, gstart_j, bj, bo):
            if swait_j is not None:
                swait(swait_j, bo)
            if gstart_j is not None:
                gstart(gstart_j, bo)
            gwait(j, bj)
            sstart(j, bj)

        del step
        gstart(0, rows[0])

        @pl.loop(0, cnt // 2 - 1)
        def _(t):
            j = 2 * t
            gwait(j, rows[0])
            gstart(j + 1, rows[1])
            sstart(j, rows[0])
            swait(j, rows[0])
            gwait(j + 1, rows[1])
            gstart(j + 2, rows[0])
            sstart(j + 1, rows[1])
            swait(j + 1, rows[1])

        gwait(cnt - 2, rows[0])
        gstart(cnt - 1, rows[1])
        sstart(cnt - 2, rows[0])
        swait(cnt - 2, rows[0])
        gwait(cnt - 1, rows[1])
        sstart(cnt - 1, rows[1])
        swait(cnt - 1, rows[1])

    with jax.named_scope("edges"):
        if NCH0 > 0:
            @pl.when(c == 0)
            def _():
                run(srcc0, dstc0, NCH0)

        if NCH1 > 0:
            @pl.when(c == 1)
            def _():
                run(srcc1, dstc1, NCH1)

        plsc.subcore_barrier()

    with jax.named_scope("writeback"):
        pltpu.sync_copy(acc.at[pl.ds(base, SPAN)],
                        out.at[c, pl.ds(base, SPAN)])


def _k1_body(d0_ref, d1_ref, x_ref, w1_ref, hwp_ref, dinv_ref):
    d = d0_ref[...].reshape(BLK, 1) + d1_ref[...].reshape(BLK, 1) + 1.0
    dinv = lax.rsqrt(d)
    hw = jnp.dot(x_ref[...], w1_ref[...], preferred_element_type=jnp.float32)
    hwp_ref[...] = hw * dinv
    dinv_ref[...] = dinv


def _k2_body(p0_ref, p1_ref, hwp_ref, dinv_ref, b1_ref, w2_ref, out_ref):
    p = p0_ref[...].reshape(BLK, H) + p1_ref[...].reshape(BLK, H)
    dinv = dinv_ref[...]
    h = jax.nn.relu(dinv * (p + hwp_ref[...]) + b1_ref[...])
    out_ref[...] = dinv * jnp.dot(h, w2_ref[...],
                                  preferred_element_type=jnp.float32)


def _k3_body(q0_ref, q1_ref, hwp_ref, dinv_ref, b2_ref, out_ref):
    q = q0_ref[...].reshape(BLK, H) + q1_ref[...].reshape(BLK, H)
    out_ref[...] = dinv_ref[...] * (q + hwp_ref[...]) + b2_ref[...]


def _row_spec(width):
    return pl.BlockSpec((BLK, width), lambda i: (i, 0))


def _part_spec(width):
    def mk(part):
        return pl.BlockSpec((1, BLK, width), lambda i, p=part: (p, i, 0))
    return mk


_k1 = pl.pallas_call(
    _k1_body,
    grid=(GRID,),
    in_specs=[
        _part_spec(1)(0),
        _part_spec(1)(1),
        _row_spec(D_IN),
        pl.BlockSpec((D_IN, H), lambda i: (0, 0)),
    ],
    out_specs=[_row_spec(H), _row_spec(1)],
    out_shape=[
        jax.ShapeDtypeStruct((N, H), jnp.float32),
        jax.ShapeDtypeStruct((N, 1), jnp.float32),
    ],
)

_k2 = pl.pallas_call(
    _k2_body,
    grid=(GRID,),
    in_specs=[
        _part_spec(H)(0),
        _part_spec(H)(1),
        _row_spec(H),
        _row_spec(1),
        pl.BlockSpec((1, H), lambda i: (0, 0)),
        pl.BlockSpec((H, H), lambda i: (0, 0)),
    ],
    out_specs=_row_spec(H),
    out_shape=jax.ShapeDtypeStruct((N, H), jnp.float32),
)

_k3 = pl.pallas_call(
    _k3_body,
    grid=(GRID,),
    in_specs=[
        _part_spec(H)(0),
        _part_spec(H)(1),
        _row_spec(H),
        _row_spec(1),
        pl.BlockSpec((1, H), lambda i: (0, 0)),
    ],
    out_specs=_row_spec(H),
    out_shape=jax.ShapeDtypeStruct((N, H), jnp.float32),
)


def kernel(x, edge_index, W1, b1, W2, b2):
    padp = EPADP - E
    srcf = jnp.concatenate([edge_index[0], jnp.zeros((padp,), jnp.int32)])
    dstf = jnp.concatenate([edge_index[1], jnp.full((padp,), TRASH, jnp.int32)])
    split = NS * NCH0 * PCH
    if NCH0:
        srcc0 = srcf[:split].reshape(NS, NCH0, PCH)
        dstc0 = dstf[:split].reshape(NS, NCH0, PCH)
    else:
        srcc0 = jnp.zeros((NS, 1, PCH), jnp.int32)
        dstc0 = jnp.full((NS, 1, PCH), TRASH, jnp.int32)
    if NCH1:
        srcc1 = srcf[split:].reshape(NS, NCH1, PCH)
        dstc1 = dstf[split:].reshape(NS, NCH1, PCH)
    else:
        srcc1 = jnp.zeros((NS, 1, PCH), jnp.int32)
        dstc1 = jnp.full((NS, 1, PCH), TRASH, jnp.int32)

    padd = EPAD - E
    dstfd = jnp.concatenate(
        [edge_index[1], jnp.full((padd,), TRASH, jnp.int32)])
    degp = _degree(dstfd.reshape(NW, NCH, CH))
    degr = degp[:, :N].reshape(NC, N, 1)
    hw1p, dinv = _k1(degr, degr, x, W1)
    p = _propagate(hw1p, srcc0, dstc0, srcc1, dstc1)
    hw2p = _k2(p, p, hw1p, dinv, b1.reshape(1, H), W2)
    q = _propagate(hw2p, srcc0, dstc0, srcc1, dstc1)
    out = _k3(q, q, hw2p, dinv, b2.reshape(1, H))
    return out


# speedup vs baseline: 1.0779x; 1.0779x over previous
import functools

import jax
import jax.numpy as jnp
from jax import lax
from jax.experimental import pallas as pl
from jax.experimental.pallas import tpu as pltpu
from jax.experimental.pallas import tpu_sc as plsc

N = 10000
E = 160000
D_IN = 256
H = 128

NC = 2
NS = 16
NW = NC * NS
CH = 128
NCH = 40
EPT = CH * NCH
EPAD = NW * EPT
SPAN = 640
ACCR = NS * SPAN
TRASH = N

BLK = 1000
GRID = N // BLK

_sc_mesh = plsc.VectorSubcoreMesh(core_axis_name="c", subcore_axis_name="s")


@functools.partial(
    pl.kernel,
    out_type=jax.ShapeDtypeStruct((NC, ACCR), jnp.float32),
    mesh=_sc_mesh,
    scratch_types=[
        pltpu.VMEM((NCH, CH), jnp.int32),
        pltpu.VMEM((SPAN,), jnp.float32),
        pltpu.VMEM((CH,), jnp.float32),
        pltpu.VMEM_SHARED((ACCR,), jnp.float32),
    ],
)
def _degree(dstp, out, dstv, buf, ones, deg):
    c = lax.axis_index("c")
    s = lax.axis_index("s")
    w = c * NS + s

    for i in range(SPAN // 16):
        buf[pl.ds(i * 16, 16)] = jnp.zeros((16,), jnp.float32)
    for i in range(CH // 16):
        ones[pl.ds(i * 16, 16)] = jnp.ones((16,), jnp.float32)

    base = s * SPAN
    pltpu.sync_copy(buf, deg.at[pl.ds(base, SPAN)])
    plsc.subcore_barrier()

    pltpu.sync_copy(dstp.at[w], dstv)

    @pl.loop(0, NCH)
    def _(j):
        pltpu.sync_copy(ones, deg.at[dstv.at[j]], add=True)

    plsc.subcore_barrier()
    pltpu.sync_copy(deg.at[pl.ds(base, SPAN)], buf)
    pltpu.sync_copy(buf, out.at[c, pl.ds(base, SPAN)])


@functools.partial(
    pl.kernel,
    out_type=jax.ShapeDtypeStruct((NC, ACCR, H), jnp.float32),
    mesh=_sc_mesh,
    scratch_types=[
        pltpu.VMEM((NCH, CH), jnp.int32),
        pltpu.VMEM((NCH, CH), jnp.int32),
        pltpu.VMEM((CH, H), jnp.float32),
        pltpu.VMEM((CH, H), jnp.float32),
        pltpu.VMEM_SHARED((ACCR, H), jnp.float32),
        pltpu.SemaphoreType.DMA,
    ],
)
def _propagate(hw, srcp, dstp, out, srcv, dstv, rows0, rows1, acc, sem):
    c = lax.axis_index("c")
    s = lax.axis_index("s")
    w = c * NS + s

    @pl.loop(0, CH)
    def _(r):
        for k in range(H // 16):
            rows0[r, pl.ds(k * 16, 16)] = jnp.zeros((16,), jnp.float32)

    base = s * SPAN
    for k in range(SPAN // CH):
        pltpu.sync_copy(rows0, acc.at[pl.ds(base + k * CH, CH)])
    plsc.subcore_barrier()

    pltpu.sync_copy(srcp.at[w], srcv)
    pltpu.sync_copy(dstp.at[w], dstv)

    def gather(j, buf):
        return pltpu.async_copy(hw.at[srcv.at[j]], buf, sem)

    def gwait(j, buf):
        pltpu.make_async_copy(hw.at[srcv.at[j]], buf, sem).wait()

    def scat(j, buf):
        pltpu.sync_copy(buf, acc.at[dstv.at[j]], add=True)

    gather(0, rows0)

    @pl.loop(0, NCH // 2 - 1)
    def _(t):
        j = 2 * t
        gwait(j, rows0)
        gather(j + 1, rows1)
        scat(j, rows0)
        gwait(j + 1, rows1)
        gather(j + 2, rows0)
        scat(j + 1, rows1)

    gwait(NCH - 2, rows0)
    gather(NCH - 1, rows1)
    scat(NCH - 2, rows0)
    gwait(NCH - 1, rows1)
    scat(NCH - 1, rows1)

    plsc.subcore_barrier()
    for k in range(SPAN // CH):
        pltpu.sync_copy(acc.at[pl.ds(base + k * CH, CH)], rows0)
        pltpu.sync_copy(rows0, out.at[c, pl.ds(base + k * CH, CH)])


def _k1_body(d0_ref, d1_ref, x_ref, w1_ref, hwp_ref, dinv_ref):
    d = d0_ref[...].reshape(BLK, 1) + d1_ref[...].reshape(BLK, 1) + 1.0
    dinv = lax.rsqrt(d)
    hw = jnp.dot(x_ref[...], w1_ref[...], preferred_element_type=jnp.float32)
    hwp_ref[...] = hw * dinv
    dinv_ref[...] = dinv


def _k2_body(p0_ref, p1_ref, hwp_ref, dinv_ref, b1_ref, w2_ref, out_ref):
    p = p0_ref[...].reshape(BLK, H) + p1_ref[...].reshape(BLK, H)
    dinv = dinv_ref[...]
    h = jax.nn.relu(dinv * (p + hwp_ref[...]) + b1_ref[...])
    out_ref[...] = dinv * jnp.dot(h, w2_ref[...],
                                  preferred_element_type=jnp.float32)


def _k3_body(q0_ref, q1_ref, hwp_ref, dinv_ref, b2_ref, out_ref):
    q = q0_ref[...].reshape(BLK, H) + q1_ref[...].reshape(BLK, H)
    out_ref[...] = dinv_ref[...] * (q + hwp_ref[...]) + b2_ref[...]


def _row_spec(width):
    return pl.BlockSpec((BLK, width), lambda i: (i, 0))


def _part_spec(width):
    def mk(part):
        return pl.BlockSpec((1, BLK, width), lambda i, p=part: (p, i, 0))
    return mk


_k1 = pl.pallas_call(
    _k1_body,
    grid=(GRID,),
    in_specs=[
        _part_spec(1)(0),
        _part_spec(1)(1),
        _row_spec(D_IN),
        pl.BlockSpec((D_IN, H), lambda i: (0, 0)),
    ],
    out_specs=[_row_spec(H), _row_spec(1)],
    out_shape=[
        jax.ShapeDtypeStruct((N, H), jnp.float32),
        jax.ShapeDtypeStruct((N, 1), jnp.float32),
    ],
)

_k2 = pl.pallas_call(
    _k2_body,
    grid=(GRID,),
    in_specs=[
        _part_spec(H)(0),
        _part_spec(H)(1),
        _row_spec(H),
        _row_spec(1),
        pl.BlockSpec((1, H), lambda i: (0, 0)),
        pl.BlockSpec((H, H), lambda i: (0, 0)),
    ],
    out_specs=_row_spec(H),
    out_shape=jax.ShapeDtypeStruct((N, H), jnp.float32),
)

_k3 = pl.pallas_call(
    _k3_body,
    grid=(GRID,),
    in_specs=[
        _part_spec(H)(0),
        _part_spec(H)(1),
        _row_spec(H),
        _row_spec(1),
        pl.BlockSpec((1, H), lambda i: (0, 0)),
    ],
    out_specs=_row_spec(H),
    out_shape=jax.ShapeDtypeStruct((N, H), jnp.float32),
)


def kernel(x, edge_index, W1, b1, W2, b2):
    pad = EPAD - E
    srcp = jnp.concatenate(
        [edge_index[0], jnp.zeros((pad,), jnp.int32)]).reshape(NW, NCH, CH)
    dstp = jnp.concatenate(
        [edge_index[1], jnp.full((pad,), TRASH, jnp.int32)]).reshape(NW, NCH, CH)

    degp = _degree(dstp)
    degr = degp[:, :N].reshape(NC, N, 1)
    hw1p, dinv = _k1(degr, degr, x, W1)
    p = _propagate(hw1p, srcp, dstp)
    hw2p = _k2(p, p, hw1p, dinv, b1.reshape(1, H), W2)
    q = _propagate(hw2p, srcp, dstp)
    out = _k3(q, q, hw2p, dinv, b2.reshape(1, H))
    return out


# speedup vs baseline: 1.0796x; 1.0016x over previous
import functools

import jax
import jax.numpy as jnp
from jax import lax
from jax.experimental import pallas as pl
from jax.experimental.pallas import tpu as pltpu
from jax.experimental.pallas import tpu_sc as plsc

N = 10000
E = 160000
D_IN = 256
H = 128

NC = 2
NS = 16
NW = NC * NS
CH = 128
NCH = 40
EPT = CH * NCH
EPAD = NW * EPT
SPAN = 640
ACCR = NS * SPAN
TRASH = N

BLK = 1000
GRID = N // BLK

_sc_mesh = plsc.VectorSubcoreMesh(core_axis_name="c", subcore_axis_name="s")


@functools.partial(
    pl.kernel,
    out_type=jax.ShapeDtypeStruct((NC, ACCR), jnp.float32),
    mesh=_sc_mesh,
    scratch_types=[
        pltpu.VMEM((NCH, CH), jnp.int32),
        pltpu.VMEM((SPAN,), jnp.float32),
        pltpu.VMEM((CH,), jnp.float32),
        pltpu.VMEM_SHARED((ACCR,), jnp.float32),
        pltpu.SemaphoreType.DMA,
    ],
)
def _degree(dstp, out, dstv, buf, ones, deg, dsem):
    c = lax.axis_index("c")
    s = lax.axis_index("s")
    w = c * NS + s

    for i in range(SPAN // 16):
        buf[pl.ds(i * 16, 16)] = jnp.zeros((16,), jnp.float32)
    for i in range(CH // 16):
        ones[pl.ds(i * 16, 16)] = jnp.ones((16,), jnp.float32)

    base = s * SPAN
    pltpu.sync_copy(buf, deg.at[pl.ds(base, SPAN)])
    plsc.subcore_barrier()

    pltpu.sync_copy(dstp.at[w], dstv)

    @pl.loop(0, NCH)
    def _(j):
        pltpu.async_copy(ones, deg.at[dstv.at[j]], dsem, add=True)

    @pl.loop(0, NCH)
    def _(j):
        pltpu.make_async_copy(ones, deg.at[dstv.at[j]], dsem).wait()

    plsc.subcore_barrier()
    pltpu.sync_copy(deg.at[pl.ds(base, SPAN)], buf)
    pltpu.sync_copy(buf, out.at[c, pl.ds(base, SPAN)])


@functools.partial(
    pl.kernel,
    out_type=jax.ShapeDtypeStruct((NC, ACCR, H), jnp.float32),
    mesh=_sc_mesh,
    scratch_types=[
        pltpu.VMEM((NCH, CH), jnp.int32),
        pltpu.VMEM((NCH, CH), jnp.int32),
        pltpu.VMEM((CH, H), jnp.float32),
        pltpu.VMEM((CH, H), jnp.float32),
        pltpu.VMEM_SHARED((ACCR, H), jnp.float32),
        pltpu.SemaphoreType.DMA,
    ],
)
def _propagate(hw, srcp, dstp, out, srcv, dstv, rows0, rows1, acc, sem):
    c = lax.axis_index("c")
    s = lax.axis_index("s")
    w = c * NS + s

    @pl.loop(0, CH)
    def _(r):
        for k in range(H // 16):
            rows0[r, pl.ds(k * 16, 16)] = jnp.zeros((16,), jnp.float32)

    base = s * SPAN
    for k in range(SPAN // CH):
        pltpu.sync_copy(rows0, acc.at[pl.ds(base + k * CH, CH)])
    plsc.subcore_barrier()

    ia = pltpu.async_copy(srcp.at[w], srcv, sem)
    ib = pltpu.async_copy(dstp.at[w], dstv, sem)
    ia.wait()
    ib.wait()

    def gather(j, buf):
        return pltpu.async_copy(hw.at[srcv.at[j]], buf, sem)

    def gwait(j, buf):
        pltpu.make_async_copy(hw.at[srcv.at[j]], buf, sem).wait()

    def scat(j, buf):
        pltpu.sync_copy(buf, acc.at[dstv.at[j]], add=True)

    gather(0, rows0)

    @pl.loop(0, NCH // 2 - 1)
    def _(t):
        j = 2 * t
        gwait(j, rows0)
        gather(j + 1, rows1)
        scat(j, rows0)
        gwait(j + 1, rows1)
        gather(j + 2, rows0)
        scat(j + 1, rows1)

    gwait(NCH - 2, rows0)
    gather(NCH - 1, rows1)
    scat(NCH - 2, rows0)
    gwait(NCH - 1, rows1)
    scat(NCH - 1, rows1)

    plsc.subcore_barrier()
    for k in range(SPAN // CH):
        pltpu.sync_copy(acc.at[pl.ds(base + k * CH, CH)], rows0)
        pltpu.sync_copy(rows0, out.at[c, pl.ds(base + k * CH, CH)])


def _k1_body(d0_ref, d1_ref, x_ref, w1_ref, hwp_ref, dinv_ref):
    d = d0_ref[...].reshape(BLK, 1) + d1_ref[...].reshape(BLK, 1) + 1.0
    dinv = lax.rsqrt(d)
    hw = jnp.dot(x_ref[...], w1_ref[...], preferred_element_type=jnp.float32)
    hwp_ref[...] = hw * dinv
    dinv_ref[...] = dinv


def _k2_body(p0_ref, p1_ref, hwp_ref, dinv_ref, b1_ref, w2_ref, out_ref):
    p = p0_ref[...].reshape(BLK, H) + p1_ref[...].reshape(BLK, H)
    dinv = dinv_ref[...]
    h = jax.nn.relu(dinv * (p + hwp_ref[...]) + b1_ref[...])
    out_ref[...] = dinv * jnp.dot(h, w2_ref[...],
                                  preferred_element_type=jnp.float32)


def _k3_body(q0_ref, q1_ref, hwp_ref, dinv_ref, b2_ref, out_ref):
    q = q0_ref[...].reshape(BLK, H) + q1_ref[...].reshape(BLK, H)
    out_ref[...] = dinv_ref[...] * (q + hwp_ref[...]) + b2_ref[...]


def _row_spec(width):
    return pl.BlockSpec((BLK, width), lambda i: (i, 0))


def _part_spec(width):
    def mk(part):
        return pl.BlockSpec((1, BLK, width), lambda i, p=part: (p, i, 0))
    return mk


_k1 = pl.pallas_call(
    _k1_body,
    grid=(GRID,),
    in_specs=[
        _part_spec(1)(0),
        _part_spec(1)(1),
        _row_spec(D_IN),
        pl.BlockSpec((D_IN, H), lambda i: (0, 0)),
    ],
    out_specs=[_row_spec(H), _row_spec(1)],
    out_shape=[
        jax.ShapeDtypeStruct((N, H), jnp.float32),
        jax.ShapeDtypeStruct((N, 1), jnp.float32),
    ],
)

_k2 = pl.pallas_call(
    _k2_body,
    grid=(GRID,),
    in_specs=[
        _part_spec(H)(0),
        _part_spec(H)(1),
        _row_spec(H),
        _row_spec(1),
        pl.BlockSpec((1, H), lambda i: (0, 0)),
        pl.BlockSpec((H, H), lambda i: (0, 0)),
    ],
    out_specs=_row_spec(H),
    out_shape=jax.ShapeDtypeStruct((N, H), jnp.float32),
)

_k3 = pl.pallas_call(
    _k3_body,
    grid=(GRID,),
    in_specs=[
        _part_spec(H)(0),
        _part_spec(H)(1),
        _row_spec(H),
        _row_spec(1),
        pl.BlockSpec((1, H), lambda i: (0, 0)),
    ],
    out_specs=_row_spec(H),
    out_shape=jax.ShapeDtypeStruct((N, H), jnp.float32),
)


def kernel(x, edge_index, W1, b1, W2, b2):
    pad = EPAD - E
    srcp = jnp.concatenate(
        [edge_index[0], jnp.zeros((pad,), jnp.int32)]).reshape(NW, NCH, CH)
    dstp = jnp.concatenate(
        [edge_index[1], jnp.full((pad,), TRASH, jnp.int32)]).reshape(NW, NCH, CH)

    degp = _degree(dstp)
    degr = degp[:, :N].reshape(NC, N, 1)
    hw1p, dinv = _k1(degr, degr, x, W1)
    p = _propagate(hw1p, srcp, dstp)
    hw2p = _k2(p, p, hw1p, dinv, b1.reshape(1, H), W2)
    q = _propagate(hw2p, srcp, dstp)
    out = _k3(q, q, hw2p, dinv, b2.reshape(1, H))
    return out
